# baseline (speedup 1.0000x reference)
import functools

import jax
import jax.numpy as jnp
from jax import lax
from jax.experimental import pallas as pl
from jax.experimental.pallas import tpu as pltpu
from jax.experimental.pallas import tpu_sc as plsc

B = 8
H = 224
W = 224
NPTS = 2048
NTILES = 32
TPB = NTILES // B
RPT = H // TPB
L = 16


def _sc_clickmap(points_t):
    mesh = plsc.VectorSubcoreMesh(core_axis_name="c", subcore_axis_name="s")

    @functools.partial(
        pl.kernel,
        mesh=mesh,
        out_type=jax.ShapeDtypeStruct((B, 1, H, W), jnp.float32),
        scratch_types=[
            pltpu.VMEM((NPTS,), jnp.int32),
            pltpu.VMEM((NPTS,), jnp.int32),
            pltpu.VMEM((RPT, W), jnp.float32),
            pltpu.SemaphoreType.DMA,
        ],
        compiler_params=pltpu.CompilerParams(needs_layout_passes=False),
    )
    def clickmap_kernel(points_hbm, out_hbm, rows_v, cols_v, band_v, sem):
        cid = lax.axis_index("c")
        sid = lax.axis_index("s")
        wid = sid * 2 + cid
        batch = wid // TPB
        row_lo = (wid % TPB) * RPT

        cp_r = pltpu.async_copy(points_hbm.at[batch, 0], rows_v, sem)
        cp_c = pltpu.async_copy(points_hbm.at[batch, 1], cols_v, sem)

        zeros_f = jnp.zeros((L,), jnp.float32)

        @plsc.parallel_loop(0, RPT, unroll=4)
        def _zero(rr):
            for cc in range(W // L):
                band_v[rr, pl.ds(cc * L, L)] = zeros_f

        cp_r.wait()
        cp_c.wait()

        ones_f = jnp.ones((L,), jnp.float32)

        @plsc.parallel_loop(0, NPTS // L, unroll=8)
        def _scatter(j):
            r = rows_v[pl.ds(j * L, L)]
            c = cols_v[pl.ds(j * L, L)]
            lr = r - row_lo
            mask = (lr >= 0) & (lr < RPT)
            safe_r = jnp.where(mask, lr, 0)
            plsc.store_scatter(band_v, [safe_r, c], ones_f, mask=mask)

        pltpu.sync_copy(band_v, out_hbm.at[batch, 0, pl.ds(row_lo, RPT)])

    return clickmap_kernel(points_t)


def kernel(x, points):
    del x
    return _sc_clickmap(points.transpose(0, 2, 1))


# speedup vs baseline: 1.0536x; 1.0536x over previous
import functools

import jax
import jax.numpy as jnp
from jax import lax
from jax.experimental import pallas as pl
from jax.experimental.pallas import tpu as pltpu
from jax.experimental.pallas import tpu_sc as plsc

B = 8
H = 224
W = 224
NPTS = 2048
NTILES = 16
TPB = NTILES // B
RPT = H // TPB
L = 16


def _sc_clickmap(points_t):
    mesh = plsc.VectorSubcoreMesh(core_axis_name="c", subcore_axis_name="s", num_cores=1)

    @functools.partial(
        pl.kernel,
        mesh=mesh,
        out_type=jax.ShapeDtypeStruct((B, 1, H, W), jnp.float32),
        scratch_types=[
            pltpu.VMEM((NPTS,), jnp.int32),
            pltpu.VMEM((NPTS,), jnp.int32),
            pltpu.VMEM((RPT, W), jnp.float32),
            pltpu.SemaphoreType.DMA,
        ],
        compiler_params=pltpu.CompilerParams(needs_layout_passes=False),
    )
    def clickmap_kernel(points_hbm, out_hbm, rows_v, cols_v, band_v, sem):
        cid = lax.axis_index("c")
        sid = lax.axis_index("s")
        wid = sid + cid
        batch = wid // TPB
        row_lo = (wid % TPB) * RPT

        cp_r = pltpu.async_copy(points_hbm.at[batch, 0], rows_v, sem)
        cp_c = pltpu.async_copy(points_hbm.at[batch, 1], cols_v, sem)

        zeros_f = jnp.zeros((L,), jnp.float32)

        @plsc.parallel_loop(0, RPT, unroll=2)
        def _zero(rr):
            for cc in range(W // L):
                band_v[rr, pl.ds(cc * L, L)] = zeros_f

        cp_r.wait()
        cp_c.wait()

        ones_f = jnp.ones((L,), jnp.float32)

        @plsc.parallel_loop(0, NPTS // L, unroll=4)
        def _scatter(j):
            r = rows_v[pl.ds(j * L, L)]
            c = cols_v[pl.ds(j * L, L)]
            lr = r - row_lo
            mask = (lr >= 0) & (lr < RPT)
            safe_r = jnp.where(mask, lr, 0)
            plsc.store_scatter(band_v, [safe_r, c], ones_f, mask=mask)

        pltpu.sync_copy(band_v, out_hbm.at[batch, 0, pl.ds(row_lo, RPT)])

    return clickmap_kernel(points_t)


def kernel(x, points):
    del x
    return _sc_clickmap(points.transpose(0, 2, 1))
